# baseline (speedup 1.0000x reference)
import functools

import jax
import jax.numpy as jnp
from jax import lax
from jax.experimental import pallas as pl
from jax.experimental.pallas import tpu as pltpu
from jax.experimental.pallas import tpu_sc as plsc

N = 10000
E = 320000
DN = 128
DE = 16
H1 = 256
NPAD = 10240
NC, NS = 2, 16
NW = NC * NS
EW = E // NW
CH = 80
RPT = NPAD // NS
WBC = 8


def _mesh():
    return plsc.VectorSubcoreMesh(core_axis_name="c", subcore_axis_name="s",
                                  num_cores=NC, num_subcores=NS)


@functools.cache
def _sc_gather_kernel():
    return pl.kernel(
        _sc_gather_body,
        out_type=(
            jax.ShapeDtypeStruct((E, DN // 2), jnp.int32),
            jax.ShapeDtypeStruct((E, DN // 2), jnp.int32),
            jax.ShapeDtypeStruct((NC, NPAD, DN), jnp.float32),
        ),
        mesh=_mesh(),
        compiler_params=pltpu.CompilerParams(use_tc_tiling_on_sc=False),
        scratch_types=[
            pltpu.VMEM((CH,), jnp.int32),
            pltpu.VMEM((CH,), jnp.int32),
            pltpu.VMEM((CH, DN // 2), jnp.int32),
            pltpu.VMEM((CH, DN // 2), jnp.int32),
            pltpu.VMEM((CH, DN), jnp.float32),
            pltpu.VMEM((RPT // WBC, DN), jnp.float32),
            pltpu.VMEM_SHARED((NPAD, DN), jnp.float32),
            pltpu.SemaphoreType.DMA,
            pltpu.SemaphoreType.DMA,
        ],
    )


def _sc_gather_body(nodes_hbm, r_hbm, s_hbm, zeros_hbm, ones_hbm,
                    a_out, c_out, cnt_out,
                    r_idx, s_idx, a_buf, c_buf, ones_buf, wb_buf, acc,
                    sem_a, sem_c):
    cid = lax.axis_index("c")
    sid = lax.axis_index("s")
    wid = cid * NS + sid
    base = wid * EW
    tb = sid * RPT

    pltpu.sync_copy(zeros_hbm, acc.at[pl.ds(tb, RPT)])
    pltpu.sync_copy(ones_hbm, ones_buf)
    plsc.subcore_barrier()

    def body(j, carry):
        cbase = base + j * CH
        pltpu.sync_copy(r_hbm.at[pl.ds(cbase, CH)], r_idx)
        pltpu.sync_copy(s_hbm.at[pl.ds(cbase, CH)], s_idx)
        ca = pltpu.async_copy(nodes_hbm.at[r_idx], a_buf, sem_a)
        cc = pltpu.async_copy(nodes_hbm.at[s_idx], c_buf, sem_c)
        pltpu.sync_copy(ones_buf, acc.at[r_idx], add=True)
        ca.wait()
        cc.wait()
        pltpu.sync_copy(a_buf, a_out.at[pl.ds(cbase, CH)])
        pltpu.sync_copy(c_buf, c_out.at[pl.ds(cbase, CH)])
        return carry

    lax.fori_loop(0, EW // CH, body, 0)
    plsc.subcore_barrier()

    def wb(k, carry):
        r0 = tb + k * (RPT // WBC)
        pltpu.sync_copy(acc.at[pl.ds(r0, RPT // WBC)], wb_buf)
        pltpu.sync_copy(wb_buf, cnt_out.at[cid, pl.ds(r0, RPT // WBC)])
        return carry

    lax.fori_loop(0, WBC, wb, 0)


@functools.cache
def _sc_scatter_kernel():
    return pl.kernel(
        _sc_scatter_body,
        out_type=jax.ShapeDtypeStruct((NC, NPAD, DN), jnp.float32),
        mesh=_mesh(),
        scratch_types=[
            pltpu.VMEM((CH,), jnp.int32),
            pltpu.VMEM((CH, DN), jnp.float32),
            pltpu.VMEM((RPT // WBC, DN), jnp.float32),
            pltpu.VMEM_SHARED((NPAD, DN), jnp.float32),
        ],
    )


def _sc_scatter_body(e2_hbm, r_hbm, zeros_hbm, p_out,
                     r_idx, row_buf, wb_buf, acc):
    cid = lax.axis_index("c")
    sid = lax.axis_index("s")
    wid = cid * NS + sid
    base = wid * EW
    tb = sid * RPT

    pltpu.sync_copy(zeros_hbm, acc.at[pl.ds(tb, RPT)])
    plsc.subcore_barrier()

    def body(j, carry):
        cbase = base + j * CH
        pltpu.sync_copy(r_hbm.at[pl.ds(cbase, CH)], r_idx)
        pltpu.sync_copy(e2_hbm.at[pl.ds(cbase, CH)], row_buf)
        pltpu.sync_copy(row_buf, acc.at[r_idx], add=True)
        return carry

    lax.fori_loop(0, EW // CH, body, 0)
    plsc.subcore_barrier()

    def wb(k, carry):
        r0 = tb + k * (RPT // WBC)
        pltpu.sync_copy(acc.at[pl.ds(r0, RPT // WBC)], wb_buf)
        pltpu.sync_copy(wb_buf, p_out.at[cid, pl.ds(r0, RPT // WBC)])
        return carry

    lax.fori_loop(0, WBC, wb, 0)


def _mlp_body(a_ref, c_ref, e_ref, w1_ref, b1_ref, w2_ref, b2_ref,
              w3_ref, b3_ref, e2_ref, eo_ref):
    w1 = w1_ref[...]
    h = jnp.dot(a_ref[...], w1[0:DN, :], preferred_element_type=jnp.float32)
    h += jnp.dot(e_ref[...].astype(jnp.bfloat16), w1[DN:DN + DE, :],
                 preferred_element_type=jnp.float32)
    h += jnp.dot(c_ref[...], w1[DN + DE:, :], preferred_element_type=jnp.float32)
    h = jax.nn.relu(h + b1_ref[...])
    e2 = jax.nn.relu(
        jnp.dot(h.astype(jnp.bfloat16), w2_ref[...],
                preferred_element_type=jnp.float32) + b2_ref[...])
    e2_ref[...] = e2
    eo_ref[...] = jax.nn.relu(
        jnp.dot(e2.astype(jnp.bfloat16), w3_ref[...],
                preferred_element_type=jnp.float32) + b3_ref[...])


def _tc_mlp(a, c, e, w1, b1, w2, b2, w3, b3, te=1280):
    grid = E // te
    blk = lambda d: pl.BlockSpec((te, d), lambda i: (i, 0))
    full = lambda s: pl.BlockSpec(s, lambda i: (0,) * len(s))
    return pl.pallas_call(
        _mlp_body,
        grid=(grid,),
        in_specs=[
            blk(DN), blk(DN), blk(DE),
            full((DN + DE + DN, H1)), full((1, H1)),
            full((H1, DN)), full((1, DN)),
            full((DN, DE)), full((1, DE)),
        ],
        out_specs=[blk(DN), blk(DE)],
        out_shape=[
            jax.ShapeDtypeStruct((E, DN), jnp.float32),
            jax.ShapeDtypeStruct((E, DE), jnp.float32),
        ],
    )(a, c, e, w1, b1, w2, b2, w3, b3)


def _combine_body(p_ref, cnt_ref, o_ref):
    s = p_ref[0, 0:N, :] + p_ref[1, 0:N, :]
    cnt = cnt_ref[0, 0:N, 0:1] + cnt_ref[1, 0:N, 0:1]
    o_ref[...] = s / jnp.maximum(cnt, 1.0)


def _tc_combine(p, cnt):
    return pl.pallas_call(
        _combine_body,
        out_shape=jax.ShapeDtypeStruct((N, DN), jnp.float32),
    )(p, cnt)


def kernel(nodes, edges, senders, receivers, W1, b1, W2, b2, W3, b3):
    b = nodes.shape[0]
    nodes_flat = nodes.reshape(N, DN)
    edges_flat = edges.reshape(E, DE)
    r = receivers.reshape(E)
    s = senders.reshape(E)

    zeros = jnp.zeros((RPT, DN), jnp.float32)
    ones = jnp.ones((CH, DN), jnp.float32)
    nodes_i32 = jax.lax.bitcast_convert_type(
        nodes_flat.astype(jnp.bfloat16).reshape(N, DN // 2, 2), jnp.int32)
    a_i32, c_i32, cnt = _sc_gather_kernel()(nodes_i32, r, s, zeros, ones)
    tobf = lambda x: jax.lax.bitcast_convert_type(x, jnp.bfloat16).reshape(E, DN)
    e2, edges_out = _tc_mlp(
        tobf(a_i32), tobf(c_i32), edges_flat,
        W1.astype(jnp.bfloat16), b1.reshape(1, H1),
        W2.astype(jnp.bfloat16), b2.reshape(1, DN),
        W3.astype(jnp.bfloat16), b3.reshape(1, DE))
    p = _sc_scatter_kernel()(e2, r, zeros)
    nodes_out = _tc_combine(p, cnt)
    return (nodes_out.reshape(b, N, DN), edges_out.reshape(b, E, DE),
            senders, receivers)


# speedup vs baseline: 2.2397x; 2.2397x over previous
import functools

import jax
import jax.numpy as jnp
from jax import lax
from jax.experimental import pallas as pl
from jax.experimental.pallas import tpu as pltpu
from jax.experimental.pallas import tpu_sc as plsc

N = 10000
E = 320000
DN = 128
DE = 16
H1 = 256
NPAD = 10240
NC, NS = 2, 16
NW = NC * NS
EW = E // NW
CH = 80
RPT = NPAD // NS
WBC = 8


def _mesh():
    return plsc.VectorSubcoreMesh(core_axis_name="c", subcore_axis_name="s",
                                  num_cores=NC, num_subcores=NS)


@functools.cache
def _sc_gather_kernel():
    return pl.kernel(
        _sc_gather_body,
        out_type=(
            jax.ShapeDtypeStruct((E, DN), jnp.float32),
            jax.ShapeDtypeStruct((E, DN), jnp.float32),
            jax.ShapeDtypeStruct((NC, NPAD, DN), jnp.float32),
        ),
        mesh=_mesh(),
        scratch_types=[
            pltpu.VMEM((CH,), jnp.int32),
            pltpu.VMEM((CH,), jnp.int32),
            pltpu.VMEM((CH, DN), jnp.float32),
            pltpu.VMEM((CH, DN), jnp.float32),
            pltpu.VMEM((CH, DN), jnp.float32),
            pltpu.VMEM((RPT // WBC, DN), jnp.float32),
            pltpu.VMEM_SHARED((NPAD, DN), jnp.float32),
            pltpu.SemaphoreType.DMA,
            pltpu.SemaphoreType.DMA,
        ],
    )


def _sc_gather_body(nodes_hbm, r_hbm, s_hbm, zeros_hbm, ones_hbm,
                    a_out, c_out, cnt_out,
                    r_idx, s_idx, a_buf, c_buf, ones_buf, wb_buf, acc,
                    sem_a, sem_c):
    cid = lax.axis_index("c")
    sid = lax.axis_index("s")
    wid = cid * NS + sid
    base = wid * EW
    tb = sid * RPT

    pltpu.sync_copy(zeros_hbm, acc.at[pl.ds(tb, RPT)])
    pltpu.sync_copy(ones_hbm, ones_buf)
    plsc.subcore_barrier()

    def body(j, carry):
        cbase = base + j * CH
        pltpu.sync_copy(r_hbm.at[pl.ds(cbase, CH)], r_idx)
        pltpu.sync_copy(s_hbm.at[pl.ds(cbase, CH)], s_idx)
        ca = pltpu.async_copy(nodes_hbm.at[r_idx], a_buf, sem_a)
        cc = pltpu.async_copy(nodes_hbm.at[s_idx], c_buf, sem_c)
        pltpu.sync_copy(ones_buf, acc.at[r_idx], add=True)
        ca.wait()
        cc.wait()
        pltpu.sync_copy(a_buf, a_out.at[pl.ds(cbase, CH)])
        pltpu.sync_copy(c_buf, c_out.at[pl.ds(cbase, CH)])
        return carry

    lax.fori_loop(0, EW // CH, body, 0)
    plsc.subcore_barrier()

    def wb(k, carry):
        r0 = tb + k * (RPT // WBC)
        pltpu.sync_copy(acc.at[pl.ds(r0, RPT // WBC)], wb_buf)
        pltpu.sync_copy(wb_buf, cnt_out.at[cid, pl.ds(r0, RPT // WBC)])
        return carry

    lax.fori_loop(0, WBC, wb, 0)


@functools.cache
def _sc_scatter_kernel():
    return pl.kernel(
        _sc_scatter_body,
        out_type=jax.ShapeDtypeStruct((NC, NPAD, DN), jnp.float32),
        mesh=_mesh(),
        scratch_types=[
            pltpu.VMEM((CH,), jnp.int32),
            pltpu.VMEM((CH, DN), jnp.float32),
            pltpu.VMEM((RPT // WBC, DN), jnp.float32),
            pltpu.VMEM_SHARED((NPAD, DN), jnp.float32),
        ],
    )


def _sc_scatter_body(e2_hbm, r_hbm, zeros_hbm, p_out,
                     r_idx, row_buf, wb_buf, acc):
    cid = lax.axis_index("c")
    sid = lax.axis_index("s")
    wid = cid * NS + sid
    base = wid * EW
    tb = sid * RPT

    pltpu.sync_copy(zeros_hbm, acc.at[pl.ds(tb, RPT)])
    plsc.subcore_barrier()

    def body(j, carry):
        cbase = base + j * CH
        pltpu.sync_copy(r_hbm.at[pl.ds(cbase, CH)], r_idx)
        pltpu.sync_copy(e2_hbm.at[pl.ds(cbase, CH)], row_buf)
        pltpu.sync_copy(row_buf, acc.at[r_idx], add=True)
        return carry

    lax.fori_loop(0, EW // CH, body, 0)
    plsc.subcore_barrier()

    def wb(k, carry):
        r0 = tb + k * (RPT // WBC)
        pltpu.sync_copy(acc.at[pl.ds(r0, RPT // WBC)], wb_buf)
        pltpu.sync_copy(wb_buf, p_out.at[cid, pl.ds(r0, RPT // WBC)])
        return carry

    lax.fori_loop(0, WBC, wb, 0)


def _mlp_body(a_ref, c_ref, e_ref, w1_ref, b1_ref, w2_ref, b2_ref,
              w3_ref, b3_ref, e2_ref, eo_ref):
    w1 = w1_ref[...]
    h = jnp.dot(a_ref[...].astype(jnp.bfloat16), w1[0:DN, :],
                preferred_element_type=jnp.float32)
    h += jnp.dot(e_ref[...].astype(jnp.bfloat16), w1[DN:DN + DE, :],
                 preferred_element_type=jnp.float32)
    h += jnp.dot(c_ref[...].astype(jnp.bfloat16), w1[DN + DE:, :],
                 preferred_element_type=jnp.float32)
    h = jax.nn.relu(h + b1_ref[...])
    e2 = jax.nn.relu(
        jnp.dot(h.astype(jnp.bfloat16), w2_ref[...],
                preferred_element_type=jnp.float32) + b2_ref[...])
    e2_ref[...] = e2
    eo_ref[...] = jax.nn.relu(
        jnp.dot(e2.astype(jnp.bfloat16), w3_ref[...],
                preferred_element_type=jnp.float32) + b3_ref[...])


def _tc_mlp(a, c, e, w1, b1, w2, b2, w3, b3, te=1280):
    grid = E // te
    blk = lambda d: pl.BlockSpec((te, d), lambda i: (i, 0))
    full = lambda s: pl.BlockSpec(s, lambda i: (0,) * len(s))
    return pl.pallas_call(
        _mlp_body,
        grid=(grid,),
        in_specs=[
            blk(DN), blk(DN), blk(DE),
            full((DN + DE + DN, H1)), full((1, H1)),
            full((H1, DN)), full((1, DN)),
            full((DN, DE)), full((1, DE)),
        ],
        out_specs=[blk(DN), blk(DE)],
        out_shape=[
            jax.ShapeDtypeStruct((E, DN), jnp.float32),
            jax.ShapeDtypeStruct((E, DE), jnp.float32),
        ],
    )(a, c, e, w1, b1, w2, b2, w3, b3)


def _combine_body(p_ref, cnt_ref, o_ref):
    s = p_ref[0, 0:N, :] + p_ref[1, 0:N, :]
    cnt = cnt_ref[0, 0:N, 0:1] + cnt_ref[1, 0:N, 0:1]
    o_ref[...] = s / jnp.maximum(cnt, 1.0)


def _tc_combine(p, cnt):
    return pl.pallas_call(
        _combine_body,
        out_shape=jax.ShapeDtypeStruct((N, DN), jnp.float32),
    )(p, cnt)


def kernel(nodes, edges, senders, receivers, W1, b1, W2, b2, W3, b3):
    b = nodes.shape[0]
    nodes_flat = nodes.reshape(N, DN)
    edges_flat = edges.reshape(E, DE)
    r = receivers.reshape(E)
    s = senders.reshape(E)

    zeros = jnp.zeros((RPT, DN), jnp.float32)
    ones = jnp.ones((CH, DN), jnp.float32)
    a_gath, c_gath, cnt = _sc_gather_kernel()(nodes_flat, r, s, zeros, ones)
    e2, edges_out = _tc_mlp(
        a_gath, c_gath, edges_flat,
        W1.astype(jnp.bfloat16), b1.reshape(1, H1),
        W2.astype(jnp.bfloat16), b2.reshape(1, DN),
        W3.astype(jnp.bfloat16), b3.reshape(1, DE))
    p = _sc_scatter_kernel()(e2, r, zeros)
    nodes_out = _tc_combine(p, cnt)
    return (nodes_out.reshape(b, N, DN), edges_out.reshape(b, E, DE),
            senders, receivers)


# speedup vs baseline: 2.5169x; 1.1238x over previous
import functools

import jax
import jax.numpy as jnp
from jax import lax
from jax.experimental import pallas as pl
from jax.experimental.pallas import tpu as pltpu
from jax.experimental.pallas import tpu_sc as plsc

N = 10000
E = 320000
DN = 128
DE = 16
H1 = 256
NPAD = 10240
NC, NS = 2, 16
NW = NC * NS
NCH = 5
ECH = E // NCH
EWC = ECH // NW
EW = E // NW
CH = 80
RPT = NPAD // NS
WBC = 8


def _mesh():
    return plsc.VectorSubcoreMesh(core_axis_name="c", subcore_axis_name="s",
                                  num_cores=NC, num_subcores=NS)


@functools.cache
def _sc_counts_kernel():
    return pl.kernel(
        _sc_counts_body,
        out_type=jax.ShapeDtypeStruct((NC, NPAD, DN), jnp.float32),
        mesh=_mesh(),
        scratch_types=[
            pltpu.VMEM((CH,), jnp.int32),
            pltpu.VMEM((CH, DN), jnp.float32),
            pltpu.VMEM((RPT // WBC, DN), jnp.float32),
            pltpu.VMEM_SHARED((NPAD, DN), jnp.float32),
        ],
    )


def _sc_counts_body(r_hbm, zeros_hbm, ones_hbm, cnt_out,
                    r_idx, ones_buf, wb_buf, acc):
    cid = lax.axis_index("c")
    sid = lax.axis_index("s")
    wid = cid * NS + sid
    base = wid * EW
    tb = sid * RPT

    pltpu.sync_copy(zeros_hbm, acc.at[pl.ds(tb, RPT)])
    pltpu.sync_copy(ones_hbm, ones_buf)
    plsc.subcore_barrier()

    def body(j, carry):
        pltpu.sync_copy(r_hbm.at[pl.ds(base + j * CH, CH)], r_idx)
        pltpu.sync_copy(ones_buf, acc.at[r_idx], add=True)
        return carry

    lax.fori_loop(0, EW // CH, body, 0)
    plsc.subcore_barrier()

    def wb(k, carry):
        r0 = tb + k * (RPT // WBC)
        pltpu.sync_copy(acc.at[pl.ds(r0, RPT // WBC)], wb_buf)
        pltpu.sync_copy(wb_buf, cnt_out.at[cid, pl.ds(r0, RPT // WBC)])
        return carry

    lax.fori_loop(0, WBC, wb, 0)


@functools.cache
def _sc_gather_kernel():
    return pl.kernel(
        _sc_gather_body,
        out_type=(
            jax.ShapeDtypeStruct((ECH, DN), jnp.float32),
            jax.ShapeDtypeStruct((ECH, DN), jnp.float32),
        ),
        mesh=_mesh(),
        scratch_types=[
            pltpu.VMEM((CH,), jnp.int32),
            pltpu.VMEM((CH,), jnp.int32),
            pltpu.VMEM((CH, DN), jnp.float32),
            pltpu.VMEM((CH, DN), jnp.float32),
            pltpu.SemaphoreType.DMA,
            pltpu.SemaphoreType.DMA,
        ],
    )


def _sc_gather_body(nodes_hbm, r_hbm, s_hbm, a_out, c_out,
                    r_idx, s_idx, a_buf, c_buf, sem_a, sem_c):
    cid = lax.axis_index("c")
    sid = lax.axis_index("s")
    wid = cid * NS + sid
    base = wid * EWC

    def body(j, carry):
        cbase = base + j * CH
        pltpu.sync_copy(r_hbm.at[pl.ds(cbase, CH)], r_idx)
        pltpu.sync_copy(s_hbm.at[pl.ds(cbase, CH)], s_idx)
        ca = pltpu.async_copy(nodes_hbm.at[r_idx], a_buf, sem_a)
        cc = pltpu.async_copy(nodes_hbm.at[s_idx], c_buf, sem_c)
        ca.wait()
        cc.wait()
        pltpu.sync_copy(a_buf, a_out.at[pl.ds(cbase, CH)])
        pltpu.sync_copy(c_buf, c_out.at[pl.ds(cbase, CH)])
        return carry

    lax.fori_loop(0, EWC // CH, body, 0)


@functools.cache
def _sc_scatter_kernel():
    return pl.kernel(
        _sc_scatter_body,
        out_type=jax.ShapeDtypeStruct((NC, NPAD, DN), jnp.float32),
        mesh=_mesh(),
        scratch_types=[
            pltpu.VMEM((CH,), jnp.int32),
            pltpu.VMEM((CH, DN), jnp.float32),
            pltpu.VMEM((RPT // WBC, DN), jnp.float32),
            pltpu.VMEM_SHARED((NPAD, DN), jnp.float32),
        ],
    )


def _sc_scatter_body(e20, e21, e22, e23, e24, r_hbm, zeros_hbm, p_out,
                     r_idx, row_buf, wb_buf, acc):
    cid = lax.axis_index("c")
    sid = lax.axis_index("s")
    wid = cid * NS + sid
    tb = sid * RPT

    pltpu.sync_copy(zeros_hbm, acc.at[pl.ds(tb, RPT)])
    plsc.subcore_barrier()

    for k, e2_hbm in enumerate((e20, e21, e22, e23, e24)):
        gbase = k * ECH + wid * EWC

        def body(j, carry, e2_hbm=e2_hbm, gbase=gbase):
            pltpu.sync_copy(r_hbm.at[pl.ds(gbase + j * CH, CH)], r_idx)
            pltpu.sync_copy(e2_hbm.at[pl.ds(wid * EWC + j * CH, CH)], row_buf)
            pltpu.sync_copy(row_buf, acc.at[r_idx], add=True)
            return carry

        lax.fori_loop(0, EWC // CH, body, 0)
    plsc.subcore_barrier()

    def wb(k, carry):
        r0 = tb + k * (RPT // WBC)
        pltpu.sync_copy(acc.at[pl.ds(r0, RPT // WBC)], wb_buf)
        pltpu.sync_copy(wb_buf, p_out.at[cid, pl.ds(r0, RPT // WBC)])
        return carry

    lax.fori_loop(0, WBC, wb, 0)


def _mlp_body(a_ref, c_ref, e_ref, w1_ref, b1_ref, w2_ref, b2_ref,
              w3_ref, b3_ref, e2_ref, eo_ref):
    w1 = w1_ref[...]
    h = jnp.dot(a_ref[...].astype(jnp.bfloat16), w1[0:DN, :],
                preferred_element_type=jnp.float32)
    h += jnp.dot(e_ref[...].astype(jnp.bfloat16), w1[DN:DN + DE, :],
                 preferred_element_type=jnp.float32)
    h += jnp.dot(c_ref[...].astype(jnp.bfloat16), w1[DN + DE:, :],
                 preferred_element_type=jnp.float32)
    h = jax.nn.relu(h + b1_ref[...])
    e2 = jax.nn.relu(
        jnp.dot(h.astype(jnp.bfloat16), w2_ref[...],
                preferred_element_type=jnp.float32) + b2_ref[...])
    e2_ref[...] = e2
    eo_ref[...] = jax.nn.relu(
        jnp.dot(e2.astype(jnp.bfloat16), w3_ref[...],
                preferred_element_type=jnp.float32) + b3_ref[...])


def _tc_mlp(a, c, e, w1, b1, w2, b2, w3, b3, te=1280):
    grid = ECH // te
    blk = lambda d: pl.BlockSpec((te, d), lambda i: (i, 0))
    full = lambda s: pl.BlockSpec(s, lambda i: (0,) * len(s))
    return pl.pallas_call(
        _mlp_body,
        grid=(grid,),
        in_specs=[
            blk(DN), blk(DN), blk(DE),
            full((DN + DE + DN, H1)), full((1, H1)),
            full((H1, DN)), full((1, DN)),
            full((DN, DE)), full((1, DE)),
        ],
        out_specs=[blk(DN), blk(DE)],
        out_shape=[
            jax.ShapeDtypeStruct((ECH, DN), jnp.float32),
            jax.ShapeDtypeStruct((ECH, DE), jnp.float32),
        ],
    )(a, c, e, w1, b1, w2, b2, w3, b3)


def _combine_body(p_ref, cnt_ref, o_ref):
    s = p_ref[0, 0:N, :] + p_ref[1, 0:N, :]
    cnt = cnt_ref[0, 0:N, 0:1] + cnt_ref[1, 0:N, 0:1]
    o_ref[...] = s / jnp.maximum(cnt, 1.0)


def _tc_combine(p, cnt):
    return pl.pallas_call(
        _combine_body,
        out_shape=jax.ShapeDtypeStruct((N, DN), jnp.float32),
    )(p, cnt)


def kernel(nodes, edges, senders, receivers, W1, b1, W2, b2, W3, b3):
    b = nodes.shape[0]
    nodes_flat = nodes.reshape(N, DN)
    edges_flat = edges.reshape(E, DE)
    r = receivers.reshape(E)
    s = senders.reshape(E)

    zeros = jnp.zeros((RPT, DN), jnp.float32)
    ones = jnp.ones((CH, DN), jnp.float32)
    cnt = _sc_counts_kernel()(r, zeros, ones)

    w1b = W1.astype(jnp.bfloat16)
    w2b = W2.astype(jnp.bfloat16)
    w3b = W3.astype(jnp.bfloat16)
    b1r, b2r, b3r = b1.reshape(1, H1), b2.reshape(1, DN), b3.reshape(1, DE)

    e2s, eos = [], []
    for k in range(NCH):
        sl = slice(k * ECH, (k + 1) * ECH)
        a_k, c_k = _sc_gather_kernel()(nodes_flat, r[sl], s[sl])
        e2_k, eo_k = _tc_mlp(a_k, c_k, edges_flat[sl],
                             w1b, b1r, w2b, b2r, w3b, b3r)
        e2s.append(e2_k)
        eos.append(eo_k)
    edges_out = jnp.concatenate(eos, axis=0)

    p = _sc_scatter_kernel()(*e2s, r, zeros)
    nodes_out = _tc_combine(p, cnt)
    return (nodes_out.reshape(b, N, DN), edges_out.reshape(b, E, DE),
            senders, receivers)


# speedup vs baseline: 2.6119x; 1.0377x over previous
import functools

import jax
import jax.numpy as jnp
from jax import lax
from jax.experimental import pallas as pl
from jax.experimental.pallas import tpu as pltpu
from jax.experimental.pallas import tpu_sc as plsc

N = 10000
E = 320000
DN = 128
DE = 16
H1 = 256
NPAD = 10240
NC, NS = 2, 16
NW = NC * NS
NCH = 5
ECH = E // NCH
EWC = ECH // NW
EW = E // NW
CH = 80
RPT = NPAD // NS
WBC = 8


def _mesh():
    return plsc.VectorSubcoreMesh(core_axis_name="c", subcore_axis_name="s",
                                  num_cores=NC, num_subcores=NS)


@functools.cache
def _sc_counts_kernel():
    return pl.kernel(
        _sc_counts_body,
        out_type=jax.ShapeDtypeStruct((NC, NPAD, DN), jnp.float32),
        mesh=_mesh(),
        scratch_types=[
            pltpu.VMEM((CH,), jnp.int32),
            pltpu.VMEM((CH, DN), jnp.float32),
            pltpu.VMEM((RPT // WBC, DN), jnp.float32),
            pltpu.VMEM_SHARED((NPAD, DN), jnp.float32),
        ],
    )


def _sc_counts_body(r_hbm, zeros_hbm, ones_hbm, cnt_out,
                    r_idx, ones_buf, wb_buf, acc):
    cid = lax.axis_index("c")
    sid = lax.axis_index("s")
    wid = cid * NS + sid
    base = wid * EW
    tb = sid * RPT

    pltpu.sync_copy(zeros_hbm, acc.at[pl.ds(tb, RPT)])
    pltpu.sync_copy(ones_hbm, ones_buf)
    plsc.subcore_barrier()

    def body(j, carry):
        pltpu.sync_copy(r_hbm.at[pl.ds(base + j * CH, CH)], r_idx)
        pltpu.sync_copy(ones_buf, acc.at[r_idx], add=True)
        return carry

    lax.fori_loop(0, EW // CH, body, 0)
    plsc.subcore_barrier()

    def wb(k, carry):
        r0 = tb + k * (RPT // WBC)
        pltpu.sync_copy(acc.at[pl.ds(r0, RPT // WBC)], wb_buf)
        pltpu.sync_copy(wb_buf, cnt_out.at[cid, pl.ds(r0, RPT // WBC)])
        return carry

    lax.fori_loop(0, WBC, wb, 0)


@functools.cache
def _sc_gather_kernel():
    return pl.kernel(
        _sc_gather_body,
        out_type=(
            jax.ShapeDtypeStruct((ECH, DN), jnp.float32),
            jax.ShapeDtypeStruct((ECH, DN), jnp.float32),
        ),
        mesh=_mesh(),
        scratch_types=[
            pltpu.VMEM((CH,), jnp.int32),
            pltpu.VMEM((CH,), jnp.int32),
            pltpu.VMEM((CH,), jnp.int32),
            pltpu.VMEM((CH,), jnp.int32),
            pltpu.VMEM((CH, DN), jnp.float32),
            pltpu.VMEM((CH, DN), jnp.float32),
            pltpu.VMEM((CH, DN), jnp.float32),
            pltpu.VMEM((CH, DN), jnp.float32),
            pltpu.SemaphoreType.DMA,
            pltpu.SemaphoreType.DMA,
            pltpu.SemaphoreType.DMA,
            pltpu.SemaphoreType.DMA,
        ],
    )


def _sc_gather_body(nodes_hbm, r_hbm, s_hbm, a_out, c_out,
                    ri0, si0, ri1, si1, a0, c0, a1, c1, sg0, sg1, sw0, sw1):
    cid = lax.axis_index("c")
    sid = lax.axis_index("s")
    wid = cid * NS + sid
    base = wid * EWC
    nch = EWC // CH

    def start_g(j, ri, si, ab, cb, sg):
        off = base + j * CH
        pltpu.sync_copy(r_hbm.at[pl.ds(off, CH)], ri)
        pltpu.sync_copy(s_hbm.at[pl.ds(off, CH)], si)
        pltpu.async_copy(nodes_hbm.at[ri], ab, sg)
        pltpu.async_copy(nodes_hbm.at[si], cb, sg)

    def wait_g(ri, si, ab, cb, sg):
        pltpu.make_async_copy(nodes_hbm.at[ri], ab, sg).wait()
        pltpu.make_async_copy(nodes_hbm.at[si], cb, sg).wait()

    def start_w(j, ab, cb, sw):
        off = base + j * CH
        pltpu.async_copy(ab, a_out.at[pl.ds(off, CH)], sw)
        pltpu.async_copy(cb, c_out.at[pl.ds(off, CH)], sw)

    def wait_w(ab, cb, sw):
        pltpu.make_async_copy(ab, a_out.at[pl.ds(0, CH)], sw).wait()
        pltpu.make_async_copy(cb, c_out.at[pl.ds(0, CH)], sw).wait()

    start_g(0, ri0, si0, a0, c0, sg0)
    start_g(1, ri1, si1, a1, c1, sg1)

    def body(t, carry):
        j = 2 * t
        wait_g(ri0, si0, a0, c0, sg0)
        start_w(j, a0, c0, sw0)
        wait_g(ri1, si1, a1, c1, sg1)
        start_w(j + 1, a1, c1, sw1)
        wait_w(a0, c0, sw0)
        start_g(j + 2, ri0, si0, a0, c0, sg0)
        wait_w(a1, c1, sw1)

        @pl.when(t < (nch - 1) // 2 - 1)
        def _():
            start_g(j + 3, ri1, si1, a1, c1, sg1)

        return carry

    lax.fori_loop(0, (nch - 1) // 2, body, 0)
    wait_g(ri0, si0, a0, c0, sg0)
    start_w(nch - 1, a0, c0, sw0)
    wait_w(a0, c0, sw0)


@functools.cache
def _sc_scatter_kernel():
    return pl.kernel(
        _sc_scatter_body,
        out_type=jax.ShapeDtypeStruct((NC, NPAD, DN), jnp.float32),
        mesh=_mesh(),
        scratch_types=[
            pltpu.VMEM((CH,), jnp.int32),
            pltpu.VMEM((CH,), jnp.int32),
            pltpu.VMEM((CH, DN), jnp.float32),
            pltpu.VMEM((CH, DN), jnp.float32),
            pltpu.VMEM((RPT // WBC, DN), jnp.float32),
            pltpu.VMEM_SHARED((NPAD, DN), jnp.float32),
            pltpu.SemaphoreType.DMA,
            pltpu.SemaphoreType.DMA,
            pltpu.SemaphoreType.DMA,
            pltpu.SemaphoreType.DMA,
        ],
    )


def _sc_scatter_body(e20, e21, e22, e23, e24, r_hbm, zeros_hbm, p_out,
                     i0, i1, b0, b1, wb_buf, acc, sl0, sl1, sa0, sa1):
    cid = lax.axis_index("c")
    sid = lax.axis_index("s")
    wid = cid * NS + sid
    tb = sid * RPT
    nch = EWC // CH

    pltpu.sync_copy(zeros_hbm, acc.at[pl.ds(tb, RPT)])
    plsc.subcore_barrier()

    for k, e2_hbm in enumerate((e20, e21, e22, e23, e24)):
        gbase = k * ECH + wid * EWC
        lbase = wid * EWC

        def start_l(j, ib, rb, sl, e2_hbm=e2_hbm, gbase=gbase, lbase=lbase):
            pltpu.async_copy(r_hbm.at[pl.ds(gbase + j * CH, CH)], ib, sl)
            pltpu.async_copy(e2_hbm.at[pl.ds(lbase + j * CH, CH)], rb, sl)

        def wait_l(ib, rb, sl):
            pltpu.make_async_copy(r_hbm.at[pl.ds(0, CH)], ib, sl).wait()
            pltpu.make_async_copy(e20.at[pl.ds(0, CH)], rb, sl).wait()

        def start_a(ib, rb, sa):
            pltpu.async_copy(rb, acc.at[ib], sa, add=True)

        def wait_a(rb, sa):
            pltpu.make_async_copy(rb, acc.at[pl.ds(0, CH)], sa).wait()

        def body(j, carry, e2_hbm=e2_hbm, gbase=gbase, lbase=lbase):
            pltpu.sync_copy(r_hbm.at[pl.ds(gbase + j * CH, CH)], i0)
            pltpu.sync_copy(e2_hbm.at[pl.ds(lbase + j * CH, CH)], b0)
            pltpu.sync_copy(b0, acc.at[i0], add=True)
            return carry

        lax.fori_loop(0, nch, body, 0)
    plsc.subcore_barrier()

    def wb(k, carry):
        r0 = tb + k * (RPT // WBC)
        pltpu.sync_copy(acc.at[pl.ds(r0, RPT // WBC)], wb_buf)
        pltpu.sync_copy(wb_buf, p_out.at[cid, pl.ds(r0, RPT // WBC)])
        return carry

    lax.fori_loop(0, WBC, wb, 0)


def _mlp_body(a_ref, c_ref, e_ref, w1_ref, b1_ref, w2_ref, b2_ref,
              w3_ref, b3_ref, e2_ref, eo_ref):
    w1 = w1_ref[...]
    h = jnp.dot(a_ref[...].astype(jnp.bfloat16), w1[0:DN, :],
                preferred_element_type=jnp.float32)
    h += jnp.dot(e_ref[...].astype(jnp.bfloat16), w1[DN:DN + DE, :],
                 preferred_element_type=jnp.float32)
    h += jnp.dot(c_ref[...].astype(jnp.bfloat16), w1[DN + DE:, :],
                 preferred_element_type=jnp.float32)
    h = jax.nn.relu(h + b1_ref[...])
    e2 = jax.nn.relu(
        jnp.dot(h.astype(jnp.bfloat16), w2_ref[...],
                preferred_element_type=jnp.float32) + b2_ref[...])
    e2_ref[...] = e2
    eo_ref[...] = jax.nn.relu(
        jnp.dot(e2.astype(jnp.bfloat16), w3_ref[...],
                preferred_element_type=jnp.float32) + b3_ref[...])


def _tc_mlp(a, c, e, w1, b1, w2, b2, w3, b3, te=1280):
    grid = ECH // te
    blk = lambda d: pl.BlockSpec((te, d), lambda i: (i, 0))
    full = lambda s: pl.BlockSpec(s, lambda i: (0,) * len(s))
    return pl.pallas_call(
        _mlp_body,
        grid=(grid,),
        in_specs=[
            blk(DN), blk(DN), blk(DE),
            full((DN + DE + DN, H1)), full((1, H1)),
            full((H1, DN)), full((1, DN)),
            full((DN, DE)), full((1, DE)),
        ],
        out_specs=[blk(DN), blk(DE)],
        out_shape=[
            jax.ShapeDtypeStruct((ECH, DN), jnp.float32),
            jax.ShapeDtypeStruct((ECH, DE), jnp.float32),
        ],
    )(a, c, e, w1, b1, w2, b2, w3, b3)


def _combine_body(p_ref, cnt_ref, o_ref):
    s = p_ref[0, 0:N, :] + p_ref[1, 0:N, :]
    cnt = cnt_ref[0, 0:N, 0:1] + cnt_ref[1, 0:N, 0:1]
    o_ref[...] = s / jnp.maximum(cnt, 1.0)


def _tc_combine(p, cnt):
    return pl.pallas_call(
        _combine_body,
        out_shape=jax.ShapeDtypeStruct((N, DN), jnp.float32),
    )(p, cnt)


def kernel(nodes, edges, senders, receivers, W1, b1, W2, b2, W3, b3):
    b = nodes.shape[0]
    nodes_flat = nodes.reshape(N, DN)
    edges_flat = edges.reshape(E, DE)
    r = receivers.reshape(E)
    s = senders.reshape(E)

    zeros = jnp.zeros((RPT, DN), jnp.float32)
    ones = jnp.ones((CH, DN), jnp.float32)
    cnt = _sc_counts_kernel()(r, zeros, ones)

    w1b = W1.astype(jnp.bfloat16)
    w2b = W2.astype(jnp.bfloat16)
    w3b = W3.astype(jnp.bfloat16)
    b1r, b2r, b3r = b1.reshape(1, H1), b2.reshape(1, DN), b3.reshape(1, DE)

    e2s, eos = [], []
    for k in range(NCH):
        sl = slice(k * ECH, (k + 1) * ECH)
        a_k, c_k = _sc_gather_kernel()(nodes_flat, r[sl], s[sl])
        e2_k, eo_k = _tc_mlp(a_k, c_k, edges_flat[sl],
                             w1b, b1r, w2b, b2r, w3b, b3r)
        e2s.append(e2_k)
        eos.append(eo_k)
    edges_out = jnp.concatenate(eos, axis=0)

    p = _sc_scatter_kernel()(*e2s, r, zeros)
    nodes_out = _tc_combine(p, cnt)
    return (nodes_out.reshape(b, N, DN), edges_out.reshape(b, E, DE),
            senders, receivers)


# speedup vs baseline: 2.9172x; 1.1169x over previous
import functools

import jax
import jax.numpy as jnp
from jax import lax
from jax.experimental import pallas as pl
from jax.experimental.pallas import tpu as pltpu
from jax.experimental.pallas import tpu_sc as plsc

N = 10000
E = 320000
DN = 128
DE = 16
H1 = 256
NPAD = 10240
NC, NS = 2, 16
NW = NC * NS
NCH = 5
ECH = E // NCH
EWC = ECH // NW
EW = E // NW
CH = 80
RPT = NPAD // NS
WBC = 8


def _mesh():
    return plsc.VectorSubcoreMesh(core_axis_name="c", subcore_axis_name="s",
                                  num_cores=NC, num_subcores=NS)


@functools.cache
def _sc_counts_kernel():
    return pl.kernel(
        _sc_counts_body,
        out_type=jax.ShapeDtypeStruct((NC, NPAD, DN), jnp.float32),
        mesh=_mesh(),
        scratch_types=[
            pltpu.VMEM((CH,), jnp.int32),
            pltpu.VMEM((CH, DN), jnp.float32),
            pltpu.VMEM((RPT // WBC, DN), jnp.float32),
            pltpu.VMEM_SHARED((NPAD, DN), jnp.float32),
        ],
    )


def _sc_counts_body(r_hbm, zeros_hbm, ones_hbm, cnt_out,
                    r_idx, ones_buf, wb_buf, acc):
    cid = lax.axis_index("c")
    sid = lax.axis_index("s")
    wid = cid * NS + sid
    base = wid * EW
    tb = sid * RPT

    pltpu.sync_copy(zeros_hbm, acc.at[pl.ds(tb, RPT)])
    pltpu.sync_copy(ones_hbm, ones_buf)
    plsc.subcore_barrier()

    def body(j, carry):
        pltpu.sync_copy(r_hbm.at[pl.ds(base + j * CH, CH)], r_idx)
        pltpu.sync_copy(ones_buf, acc.at[r_idx], add=True)
        return carry

    lax.fori_loop(0, EW // CH, body, 0)
    plsc.subcore_barrier()

    def wb(k, carry):
        r0 = tb + k * (RPT // WBC)
        pltpu.sync_copy(acc.at[pl.ds(r0, RPT // WBC)], wb_buf)
        pltpu.sync_copy(wb_buf, cnt_out.at[cid, pl.ds(r0, RPT // WBC)])
        return carry

    lax.fori_loop(0, WBC, wb, 0)


@functools.cache
def _sc_gather_kernel():
    return pl.kernel(
        _sc_gather_body,
        out_type=(
            jax.ShapeDtypeStruct((ECH, DN), jnp.float32),
            jax.ShapeDtypeStruct((ECH, DN), jnp.float32),
        ),
        mesh=_mesh(),
        scratch_types=[
            pltpu.VMEM((CH,), jnp.int32),
            pltpu.VMEM((CH,), jnp.int32),
            pltpu.VMEM((CH,), jnp.int32),
            pltpu.VMEM((CH,), jnp.int32),
            pltpu.VMEM((CH, DN), jnp.float32),
            pltpu.VMEM((CH, DN), jnp.float32),
            pltpu.VMEM((CH, DN), jnp.float32),
            pltpu.VMEM((CH, DN), jnp.float32),
            pltpu.SemaphoreType.DMA,
            pltpu.SemaphoreType.DMA,
            pltpu.SemaphoreType.DMA,
            pltpu.SemaphoreType.DMA,
        ],
    )


def _sc_gather_body(nodes_hbm, r_hbm, s_hbm, a_out, c_out,
                    ri0, si0, ri1, si1, a0, c0, a1, c1, sg0, sg1, sw0, sw1):
    cid = lax.axis_index("c")
    sid = lax.axis_index("s")
    wid = cid * NS + sid
    base = wid * EWC
    nch = EWC // CH

    def start_g(j, ri, si, ab, cb, sg):
        off = base + j * CH
        pltpu.sync_copy(r_hbm.at[pl.ds(off, CH)], ri)
        pltpu.sync_copy(s_hbm.at[pl.ds(off, CH)], si)
        pltpu.async_copy(nodes_hbm.at[ri], ab, sg)
        pltpu.async_copy(nodes_hbm.at[si], cb, sg)

    def wait_g(ri, si, ab, cb, sg):
        pltpu.make_async_copy(nodes_hbm.at[ri], ab, sg).wait()
        pltpu.make_async_copy(nodes_hbm.at[si], cb, sg).wait()

    def start_w(j, ab, cb, sw):
        off = base + j * CH
        pltpu.async_copy(ab, a_out.at[pl.ds(off, CH)], sw)
        pltpu.async_copy(cb, c_out.at[pl.ds(off, CH)], sw)

    def wait_w(ab, cb, sw):
        pltpu.make_async_copy(ab, a_out.at[pl.ds(0, CH)], sw).wait()
        pltpu.make_async_copy(cb, c_out.at[pl.ds(0, CH)], sw).wait()

    start_g(0, ri0, si0, a0, c0, sg0)
    start_g(1, ri1, si1, a1, c1, sg1)

    def body(t, carry):
        j = 2 * t
        wait_g(ri0, si0, a0, c0, sg0)
        start_w(j, a0, c0, sw0)
        wait_g(ri1, si1, a1, c1, sg1)
        start_w(j + 1, a1, c1, sw1)
        wait_w(a0, c0, sw0)
        start_g(j + 2, ri0, si0, a0, c0, sg0)
        wait_w(a1, c1, sw1)

        @pl.when(t < (nch - 1) // 2 - 1)
        def _():
            start_g(j + 3, ri1, si1, a1, c1, sg1)

        return carry

    lax.fori_loop(0, (nch - 1) // 2, body, 0)
    wait_g(ri0, si0, a0, c0, sg0)
    start_w(nch - 1, a0, c0, sw0)
    wait_w(a0, c0, sw0)


@functools.cache
def _sc_scatter_kernel():
    return pl.kernel(
        _sc_scatter_body,
        out_type=jax.ShapeDtypeStruct((NC, NPAD, DN), jnp.float32),
        mesh=_mesh(),
        scratch_types=[
            pltpu.VMEM((CH,), jnp.int32),
            pltpu.VMEM((CH,), jnp.int32),
            pltpu.VMEM((CH, DN), jnp.float32),
            pltpu.VMEM((CH, DN), jnp.float32),
            pltpu.VMEM((RPT // WBC, DN), jnp.float32),
            pltpu.VMEM_SHARED((NPAD, DN), jnp.float32),
            pltpu.SemaphoreType.DMA,
            pltpu.SemaphoreType.DMA,
            pltpu.SemaphoreType.DMA,
            pltpu.SemaphoreType.DMA,
        ],
    )


def _sc_scatter_body(e20, e21, e22, e23, e24, r_hbm, zeros_hbm, p_out,
                     i0, i1, b0, b1, wb_buf, acc, sl0, sl1, sa0, sa1):
    cid = lax.axis_index("c")
    sid = lax.axis_index("s")
    wid = cid * NS + sid
    tb = sid * RPT
    nch = EWC // CH

    pltpu.sync_copy(zeros_hbm, acc.at[pl.ds(tb, RPT)])
    plsc.subcore_barrier()

    for k, e2_hbm in enumerate((e20, e21, e22, e23, e24)):
        gbase = k * ECH + wid * EWC
        lbase = wid * EWC

        def start_l(j, ib, rb, sl, e2_hbm=e2_hbm, gbase=gbase, lbase=lbase):
            pltpu.async_copy(r_hbm.at[pl.ds(gbase + j * CH, CH)], ib, sl)
            pltpu.async_copy(e2_hbm.at[pl.ds(lbase + j * CH, CH)], rb, sl)

        def wait_l(ib, rb, sl):
            pltpu.make_async_copy(r_hbm.at[pl.ds(0, CH)], ib, sl).wait()
            pltpu.make_async_copy(e20.at[pl.ds(0, CH)], rb, sl).wait()

        def start_a(ib, rb, sa):
            pltpu.async_copy(rb, acc.at[ib], sa, add=True)

        def wait_a(rb, sa):
            pltpu.make_async_copy(rb, acc.at[pl.ds(0, CH)], sa).wait()

        def start_l(j, ib, rb, sl, e2_hbm=e2_hbm, gbase=gbase, lbase=lbase):
            pltpu.async_copy(r_hbm.at[pl.ds(gbase + j * CH, CH)], ib, sl)
            pltpu.async_copy(e2_hbm.at[pl.ds(lbase + j * CH, CH)], rb, sl)

        def wait_l(ib, rb, sl, e2_hbm=e2_hbm, gbase=gbase, lbase=lbase):
            pltpu.make_async_copy(r_hbm.at[pl.ds(gbase, CH)], ib, sl).wait()
            pltpu.make_async_copy(e2_hbm.at[pl.ds(lbase, CH)], rb, sl).wait()

        def start_a(ib, rb, sa):
            pltpu.async_copy(rb, acc.at[ib], sa, add=True)

        def wait_a(ib, rb, sa):
            pltpu.make_async_copy(rb, acc.at[ib], sa).wait()

        start_l(0, i0, b0, sl0)
        start_l(1, i1, b1, sl1)

        def body(t, carry):
            j = 2 * t
            wait_l(i0, b0, sl0)
            start_a(i0, b0, sa0)
            wait_l(i1, b1, sl1)
            start_a(i1, b1, sa1)
            wait_a(i0, b0, sa0)
            start_l(j + 2, i0, b0, sl0)
            wait_a(i1, b1, sa1)

            @pl.when(t < (nch - 1) // 2 - 1)
            def _():
                start_l(j + 3, i1, b1, sl1)

            return carry

        lax.fori_loop(0, (nch - 1) // 2, body, 0)
        wait_l(i0, b0, sl0)
        start_a(i0, b0, sa0)
        wait_a(i0, b0, sa0)
    plsc.subcore_barrier()

    def wb(k, carry):
        r0 = tb + k * (RPT // WBC)
        pltpu.sync_copy(acc.at[pl.ds(r0, RPT // WBC)], wb_buf)
        pltpu.sync_copy(wb_buf, p_out.at[cid, pl.ds(r0, RPT // WBC)])
        return carry

    lax.fori_loop(0, WBC, wb, 0)


def _mlp_body(a_ref, c_ref, e_ref, w1_ref, b1_ref, w2_ref, b2_ref,
              w3_ref, b3_ref, e2_ref, eo_ref):
    w1 = w1_ref[...]
    h = jnp.dot(a_ref[...].astype(jnp.bfloat16), w1[0:DN, :],
                preferred_element_type=jnp.float32)
    h += jnp.dot(e_ref[...].astype(jnp.bfloat16), w1[DN:DN + DE, :],
                 preferred_element_type=jnp.float32)
    h += jnp.dot(c_ref[...].astype(jnp.bfloat16), w1[DN + DE:, :],
                 preferred_element_type=jnp.float32)
    h = jax.nn.relu(h + b1_ref[...])
    e2 = jax.nn.relu(
        jnp.dot(h.astype(jnp.bfloat16), w2_ref[...],
                preferred_element_type=jnp.float32) + b2_ref[...])
    e2_ref[...] = e2
    eo_ref[...] = jax.nn.relu(
        jnp.dot(e2.astype(jnp.bfloat16), w3_ref[...],
                preferred_element_type=jnp.float32) + b3_ref[...])


def _tc_mlp(a, c, e, w1, b1, w2, b2, w3, b3, te=1280):
    grid = ECH // te
    blk = lambda d: pl.BlockSpec((te, d), lambda i: (i, 0))
    full = lambda s: pl.BlockSpec(s, lambda i: (0,) * len(s))
    return pl.pallas_call(
        _mlp_body,
        grid=(grid,),
        in_specs=[
            blk(DN), blk(DN), blk(DE),
            full((DN + DE + DN, H1)), full((1, H1)),
            full((H1, DN)), full((1, DN)),
            full((DN, DE)), full((1, DE)),
        ],
        out_specs=[blk(DN), blk(DE)],
        out_shape=[
            jax.ShapeDtypeStruct((ECH, DN), jnp.float32),
            jax.ShapeDtypeStruct((ECH, DE), jnp.float32),
        ],
    )(a, c, e, w1, b1, w2, b2, w3, b3)


def _combine_body(p_ref, cnt_ref, o_ref):
    s = p_ref[0, 0:N, :] + p_ref[1, 0:N, :]
    cnt = cnt_ref[0, 0:N, 0:1] + cnt_ref[1, 0:N, 0:1]
    o_ref[...] = s / jnp.maximum(cnt, 1.0)


def _tc_combine(p, cnt):
    return pl.pallas_call(
        _combine_body,
        out_shape=jax.ShapeDtypeStruct((N, DN), jnp.float32),
    )(p, cnt)


def kernel(nodes, edges, senders, receivers, W1, b1, W2, b2, W3, b3):
    b = nodes.shape[0]
    nodes_flat = nodes.reshape(N, DN)
    edges_flat = edges.reshape(E, DE)
    r = receivers.reshape(E)
    s = senders.reshape(E)

    zeros = jnp.zeros((RPT, DN), jnp.float32)
    ones = jnp.ones((CH, DN), jnp.float32)
    cnt = _sc_counts_kernel()(r, zeros, ones)

    w1b = W1.astype(jnp.bfloat16)
    w2b = W2.astype(jnp.bfloat16)
    w3b = W3.astype(jnp.bfloat16)
    b1r, b2r, b3r = b1.reshape(1, H1), b2.reshape(1, DN), b3.reshape(1, DE)

    e2s, eos = [], []
    for k in range(NCH):
        sl = slice(k * ECH, (k + 1) * ECH)
        a_k, c_k = _sc_gather_kernel()(nodes_flat, r[sl], s[sl])
        e2_k, eo_k = _tc_mlp(a_k, c_k, edges_flat[sl],
                             w1b, b1r, w2b, b2r, w3b, b3r)
        e2s.append(e2_k)
        eos.append(eo_k)
    edges_out = jnp.concatenate(eos, axis=0)

    p = _sc_scatter_kernel()(*e2s, r, zeros)
    nodes_out = _tc_combine(p, cnt)
    return (nodes_out.reshape(b, N, DN), edges_out.reshape(b, E, DE),
            senders, receivers)


# speedup vs baseline: 2.9208x; 1.0012x over previous
import functools

import jax
import jax.numpy as jnp
from jax import lax
from jax.experimental import pallas as pl
from jax.experimental.pallas import tpu as pltpu
from jax.experimental.pallas import tpu_sc as plsc

N = 10000
E = 320000
DN = 128
DE = 16
H1 = 256
NPAD = 10240
NC, NS = 2, 16
NW = NC * NS
NCH = 5
ECH = E // NCH
EWC = ECH // NW
EW = E // NW
CH = 80
RPT = NPAD // NS
WBC = 8


def _mesh():
    return plsc.VectorSubcoreMesh(core_axis_name="c", subcore_axis_name="s",
                                  num_cores=NC, num_subcores=NS)


@functools.cache
def _sc_counts_kernel():
    return pl.kernel(
        _sc_counts_body,
        out_type=jax.ShapeDtypeStruct((NC, NPAD, DN), jnp.float32),
        mesh=_mesh(),
        scratch_types=[
            pltpu.VMEM((CH,), jnp.int32),
            pltpu.VMEM((CH,), jnp.int32),
            pltpu.VMEM((CH, DN), jnp.float32),
            pltpu.VMEM((RPT // WBC, DN), jnp.float32),
            pltpu.VMEM_SHARED((NPAD, DN), jnp.float32),
            pltpu.SemaphoreType.DMA,
            pltpu.SemaphoreType.DMA,
            pltpu.SemaphoreType.DMA,
            pltpu.SemaphoreType.DMA,
        ],
    )


def _sc_counts_body(r_hbm, zeros_hbm, ones_hbm, cnt_out,
                    i0, i1, ones_buf, wb_buf, acc, sl0, sl1, sa0, sa1):
    cid = lax.axis_index("c")
    sid = lax.axis_index("s")
    wid = cid * NS + sid
    base = wid * EW
    tb = sid * RPT
    nch = EW // CH

    pltpu.sync_copy(zeros_hbm, acc.at[pl.ds(tb, RPT)])
    pltpu.sync_copy(ones_hbm, ones_buf)
    plsc.subcore_barrier()

    def start_l(j, ib, sl):
        pltpu.async_copy(r_hbm.at[pl.ds(base + j * CH, CH)], ib, sl)

    def wait_l(ib, sl):
        pltpu.make_async_copy(r_hbm.at[pl.ds(base, CH)], ib, sl).wait()

    def start_a(ib, sa):
        pltpu.async_copy(ones_buf, acc.at[ib], sa, add=True)

    def wait_a(ib, sa):
        pltpu.make_async_copy(ones_buf, acc.at[ib], sa).wait()

    start_l(0, i0, sl0)
    start_l(1, i1, sl1)

    def body(t, carry):
        j = 2 * t
        wait_l(i0, sl0)
        start_a(i0, sa0)
        wait_l(i1, sl1)
        start_a(i1, sa1)
        wait_a(i0, sa0)
        start_l(j + 2, i0, sl0)
        wait_a(i1, sa1)

        @pl.when(t < (nch - 1) // 2 - 1)
        def _():
            start_l(j + 3, i1, sl1)

        return carry

    lax.fori_loop(0, (nch - 1) // 2, body, 0)
    wait_l(i0, sl0)
    start_a(i0, sa0)
    wait_a(i0, sa0)
    plsc.subcore_barrier()

    def wb(k, carry):
        r0 = tb + k * (RPT // WBC)
        pltpu.sync_copy(acc.at[pl.ds(r0, RPT // WBC)], wb_buf)
        pltpu.sync_copy(wb_buf, cnt_out.at[cid, pl.ds(r0, RPT // WBC)])
        return carry

    lax.fori_loop(0, WBC, wb, 0)


@functools.cache
def _sc_gather_kernel():
    return pl.kernel(
        _sc_gather_body,
        out_type=(
            jax.ShapeDtypeStruct((ECH, DN), jnp.float32),
            jax.ShapeDtypeStruct((ECH, DN), jnp.float32),
        ),
        mesh=_mesh(),
        scratch_types=[
            pltpu.VMEM((CH,), jnp.int32),
            pltpu.VMEM((CH,), jnp.int32),
            pltpu.VMEM((CH,), jnp.int32),
            pltpu.VMEM((CH,), jnp.int32),
            pltpu.VMEM((CH, DN), jnp.float32),
            pltpu.VMEM((CH, DN), jnp.float32),
            pltpu.VMEM((CH, DN), jnp.float32),
            pltpu.VMEM((CH, DN), jnp.float32),
            pltpu.SemaphoreType.DMA,
            pltpu.SemaphoreType.DMA,
            pltpu.SemaphoreType.DMA,
            pltpu.SemaphoreType.DMA,
        ],
    )


def _sc_gather_body(nodes_hbm, r_hbm, s_hbm, a_out, c_out,
                    ri0, si0, ri1, si1, a0, c0, a1, c1, sg0, sg1, sw0, sw1):
    cid = lax.axis_index("c")
    sid = lax.axis_index("s")
    wid = cid * NS + sid
    base = wid * EWC
    nch = EWC // CH

    def start_g(j, ri, si, ab, cb, sg):
        off = base + j * CH
        pltpu.sync_copy(r_hbm.at[pl.ds(off, CH)], ri)
        pltpu.sync_copy(s_hbm.at[pl.ds(off, CH)], si)
        pltpu.async_copy(nodes_hbm.at[ri], ab, sg)
        pltpu.async_copy(nodes_hbm.at[si], cb, sg)

    def wait_g(ri, si, ab, cb, sg):
        pltpu.make_async_copy(nodes_hbm.at[ri], ab, sg).wait()
        pltpu.make_async_copy(nodes_hbm.at[si], cb, sg).wait()

    def start_w(j, ab, cb, sw):
        off = base + j * CH
        pltpu.async_copy(ab, a_out.at[pl.ds(off, CH)], sw)
        pltpu.async_copy(cb, c_out.at[pl.ds(off, CH)], sw)

    def wait_w(ab, cb, sw):
        pltpu.make_async_copy(ab, a_out.at[pl.ds(0, CH)], sw).wait()
        pltpu.make_async_copy(cb, c_out.at[pl.ds(0, CH)], sw).wait()

    start_g(0, ri0, si0, a0, c0, sg0)
    start_g(1, ri1, si1, a1, c1, sg1)

    def body(t, carry):
        j = 2 * t
        wait_g(ri0, si0, a0, c0, sg0)
        start_w(j, a0, c0, sw0)
        wait_g(ri1, si1, a1, c1, sg1)
        start_w(j + 1, a1, c1, sw1)
        wait_w(a0, c0, sw0)
        start_g(j + 2, ri0, si0, a0, c0, sg0)
        wait_w(a1, c1, sw1)

        @pl.when(t < (nch - 1) // 2 - 1)
        def _():
            start_g(j + 3, ri1, si1, a1, c1, sg1)

        return carry

    lax.fori_loop(0, (nch - 1) // 2, body, 0)
    wait_g(ri0, si0, a0, c0, sg0)
    start_w(nch - 1, a0, c0, sw0)
    wait_w(a0, c0, sw0)


@functools.cache
def _sc_scatter_kernel():
    return pl.kernel(
        _sc_scatter_body,
        out_type=jax.ShapeDtypeStruct((NC, NPAD, DN), jnp.float32),
        mesh=_mesh(),
        scratch_types=[
            pltpu.VMEM((CH,), jnp.int32),
            pltpu.VMEM((CH,), jnp.int32),
            pltpu.VMEM((CH, DN), jnp.float32),
            pltpu.VMEM((CH, DN), jnp.float32),
            pltpu.VMEM((RPT // WBC, DN), jnp.float32),
            pltpu.VMEM_SHARED((NPAD, DN), jnp.float32),
            pltpu.SemaphoreType.DMA,
            pltpu.SemaphoreType.DMA,
            pltpu.SemaphoreType.DMA,
            pltpu.SemaphoreType.DMA,
        ],
    )


def _sc_scatter_body(e20, e21, e22, e23, e24, r_hbm, zeros_hbm, p_out,
                     i0, i1, b0, b1, wb_buf, acc, sl0, sl1, sa0, sa1):
    cid = lax.axis_index("c")
    sid = lax.axis_index("s")
    wid = cid * NS + sid
    tb = sid * RPT
    nch = EWC // CH

    pltpu.sync_copy(zeros_hbm, acc.at[pl.ds(tb, RPT)])
    plsc.subcore_barrier()

    for k, e2_hbm in enumerate((e20, e21, e22, e23, e24)):
        gbase = k * ECH + wid * EWC
        lbase = wid * EWC

        def start_l(j, ib, rb, sl, e2_hbm=e2_hbm, gbase=gbase, lbase=lbase):
            pltpu.async_copy(r_hbm.at[pl.ds(gbase + j * CH, CH)], ib, sl)
            pltpu.async_copy(e2_hbm.at[pl.ds(lbase + j * CH, CH)], rb, sl)

        def wait_l(ib, rb, sl):
            pltpu.make_async_copy(r_hbm.at[pl.ds(0, CH)], ib, sl).wait()
            pltpu.make_async_copy(e20.at[pl.ds(0, CH)], rb, sl).wait()

        def start_a(ib, rb, sa):
            pltpu.async_copy(rb, acc.at[ib], sa, add=True)

        def wait_a(rb, sa):
            pltpu.make_async_copy(rb, acc.at[pl.ds(0, CH)], sa).wait()

        def start_l(j, ib, rb, sl, e2_hbm=e2_hbm, gbase=gbase, lbase=lbase):
            pltpu.async_copy(r_hbm.at[pl.ds(gbase + j * CH, CH)], ib, sl)
            pltpu.async_copy(e2_hbm.at[pl.ds(lbase + j * CH, CH)], rb, sl)

        def wait_l(ib, rb, sl, e2_hbm=e2_hbm, gbase=gbase, lbase=lbase):
            pltpu.make_async_copy(r_hbm.at[pl.ds(gbase, CH)], ib, sl).wait()
            pltpu.make_async_copy(e2_hbm.at[pl.ds(lbase, CH)], rb, sl).wait()

        def start_a(ib, rb, sa):
            pltpu.async_copy(rb, acc.at[ib], sa, add=True)

        def wait_a(ib, rb, sa):
            pltpu.make_async_copy(rb, acc.at[ib], sa).wait()

        start_l(0, i0, b0, sl0)
        start_l(1, i1, b1, sl1)

        def body(t, carry):
            j = 2 * t
            wait_l(i0, b0, sl0)
            start_a(i0, b0, sa0)
            wait_l(i1, b1, sl1)
            start_a(i1, b1, sa1)
            wait_a(i0, b0, sa0)
            start_l(j + 2, i0, b0, sl0)
            wait_a(i1, b1, sa1)

            @pl.when(t < (nch - 1) // 2 - 1)
            def _():
                start_l(j + 3, i1, b1, sl1)

            return carry

        lax.fori_loop(0, (nch - 1) // 2, body, 0)
        wait_l(i0, b0, sl0)
        start_a(i0, b0, sa0)
        wait_a(i0, b0, sa0)
    plsc.subcore_barrier()

    def wb(k, carry):
        r0 = tb + k * (RPT // WBC)
        pltpu.sync_copy(acc.at[pl.ds(r0, RPT // WBC)], wb_buf)
        pltpu.sync_copy(wb_buf, p_out.at[cid, pl.ds(r0, RPT // WBC)])
        return carry

    lax.fori_loop(0, WBC, wb, 0)


def _mlp_body(a_ref, c_ref, e_ref, w1_ref, b1_ref, w2_ref, b2_ref,
              w3_ref, b3_ref, e2_ref, eo_ref):
    w1 = w1_ref[...]
    h = jnp.dot(a_ref[...].astype(jnp.bfloat16), w1[0:DN, :],
                preferred_element_type=jnp.float32)
    h += jnp.dot(e_ref[...].astype(jnp.bfloat16), w1[DN:DN + DE, :],
                 preferred_element_type=jnp.float32)
    h += jnp.dot(c_ref[...].astype(jnp.bfloat16), w1[DN + DE:, :],
                 preferred_element_type=jnp.float32)
    h = jax.nn.relu(h + b1_ref[...])
    e2 = jax.nn.relu(
        jnp.dot(h.astype(jnp.bfloat16), w2_ref[...],
                preferred_element_type=jnp.float32) + b2_ref[...])
    e2_ref[...] = e2
    eo_ref[...] = jax.nn.relu(
        jnp.dot(e2.astype(jnp.bfloat16), w3_ref[...],
                preferred_element_type=jnp.float32) + b3_ref[...])


def _tc_mlp(a, c, e, w1, b1, w2, b2, w3, b3, te=1280):
    grid = ECH // te
    blk = lambda d: pl.BlockSpec((te, d), lambda i: (i, 0))
    full = lambda s: pl.BlockSpec(s, lambda i: (0,) * len(s))
    return pl.pallas_call(
        _mlp_body,
        grid=(grid,),
        in_specs=[
            blk(DN), blk(DN), blk(DE),
            full((DN + DE + DN, H1)), full((1, H1)),
            full((H1, DN)), full((1, DN)),
            full((DN, DE)), full((1, DE)),
        ],
        out_specs=[blk(DN), blk(DE)],
        out_shape=[
            jax.ShapeDtypeStruct((ECH, DN), jnp.float32),
            jax.ShapeDtypeStruct((ECH, DE), jnp.float32),
        ],
    )(a, c, e, w1, b1, w2, b2, w3, b3)


def _combine_body(p_ref, cnt_ref, o_ref):
    s = p_ref[0, 0:N, :] + p_ref[1, 0:N, :]
    cnt = cnt_ref[0, 0:N, 0:1] + cnt_ref[1, 0:N, 0:1]
    o_ref[...] = s / jnp.maximum(cnt, 1.0)


def _tc_combine(p, cnt):
    return pl.pallas_call(
        _combine_body,
        out_shape=jax.ShapeDtypeStruct((N, DN), jnp.float32),
    )(p, cnt)


def kernel(nodes, edges, senders, receivers, W1, b1, W2, b2, W3, b3):
    b = nodes.shape[0]
    nodes_flat = nodes.reshape(N, DN)
    edges_flat = edges.reshape(E, DE)
    r = receivers.reshape(E)
    s = senders.reshape(E)

    zeros = jnp.zeros((RPT, DN), jnp.float32)
    ones = jnp.ones((CH, DN), jnp.float32)
    cnt = _sc_counts_kernel()(r, zeros, ones)

    w1b = W1.astype(jnp.bfloat16)
    w2b = W2.astype(jnp.bfloat16)
    w3b = W3.astype(jnp.bfloat16)
    b1r, b2r, b3r = b1.reshape(1, H1), b2.reshape(1, DN), b3.reshape(1, DE)

    e2s, eos = [], []
    for k in range(NCH):
        sl = slice(k * ECH, (k + 1) * ECH)
        a_k, c_k = _sc_gather_kernel()(nodes_flat, r[sl], s[sl])
        e2_k, eo_k = _tc_mlp(a_k, c_k, edges_flat[sl],
                             w1b, b1r, w2b, b2r, w3b, b3r)
        e2s.append(e2_k)
        eos.append(eo_k)
    edges_out = jnp.concatenate(eos, axis=0)

    p = _sc_scatter_kernel()(*e2s, r, zeros)
    nodes_out = _tc_combine(p, cnt)
    return (nodes_out.reshape(b, N, DN), edges_out.reshape(b, E, DE),
            senders, receivers)


# speedup vs baseline: 3.0429x; 1.0418x over previous
import functools

import jax
import jax.numpy as jnp
from jax import lax
from jax.experimental import pallas as pl
from jax.experimental.pallas import tpu as pltpu
from jax.experimental.pallas import tpu_sc as plsc

N = 10000
E = 320000
DN = 128
DE = 16
H1 = 256
NPAD = 10240
NC, NS = 2, 16
NW = NC * NS
NCH = 5
ECH = E // NCH
EWC = ECH // NW
EW = E // NW
CH = 80
RPT = NPAD // NS
WBC = 8


def _mesh():
    return plsc.VectorSubcoreMesh(core_axis_name="c", subcore_axis_name="s",
                                  num_cores=NC, num_subcores=NS)


@functools.cache
def _sc_counts_kernel():
    return pl.kernel(
        _sc_counts_body,
        out_type=jax.ShapeDtypeStruct((NC, NPAD, DN), jnp.float32),
        mesh=_mesh(),
        scratch_types=[
            pltpu.VMEM((CH,), jnp.int32),
            pltpu.VMEM((CH,), jnp.int32),
            pltpu.VMEM((CH, DN), jnp.float32),
            pltpu.VMEM((RPT // WBC, DN), jnp.float32),
            pltpu.VMEM_SHARED((NPAD, DN), jnp.float32),
            pltpu.SemaphoreType.DMA,
            pltpu.SemaphoreType.DMA,
            pltpu.SemaphoreType.DMA,
            pltpu.SemaphoreType.DMA,
        ],
    )


def _sc_counts_body(r_hbm, zeros_hbm, ones_hbm, cnt_out,
                    i0, i1, ones_buf, wb_buf, acc, sl0, sl1, sa0, sa1):
    cid = lax.axis_index("c")
    sid = lax.axis_index("s")
    wid = cid * NS + sid
    base = wid * EW
    tb = sid * RPT
    nch = EW // CH

    pltpu.sync_copy(zeros_hbm, acc.at[pl.ds(tb, RPT)])
    pltpu.sync_copy(ones_hbm, ones_buf)
    plsc.subcore_barrier()

    def start_l(j, ib, sl):
        pltpu.async_copy(r_hbm.at[pl.ds(base + j * CH, CH)], ib, sl)

    def wait_l(ib, sl):
        pltpu.make_async_copy(r_hbm.at[pl.ds(base, CH)], ib, sl).wait()

    def start_a(ib, sa):
        pltpu.async_copy(ones_buf, acc.at[ib], sa, add=True)

    def wait_a(ib, sa):
        pltpu.make_async_copy(ones_buf, acc.at[ib], sa).wait()

    start_l(0, i0, sl0)
    start_l(1, i1, sl1)

    def body(t, carry):
        j = 2 * t
        wait_l(i0, sl0)
        start_a(i0, sa0)
        wait_l(i1, sl1)
        start_a(i1, sa1)
        wait_a(i0, sa0)
        start_l(j + 2, i0, sl0)
        wait_a(i1, sa1)

        @pl.when(t < (nch - 1) // 2 - 1)
        def _():
            start_l(j + 3, i1, sl1)

        return carry

    lax.fori_loop(0, (nch - 1) // 2, body, 0)
    wait_l(i0, sl0)
    start_a(i0, sa0)
    wait_a(i0, sa0)
    plsc.subcore_barrier()

    def wb(k, carry):
        r0 = tb + k * (RPT // WBC)
        pltpu.sync_copy(acc.at[pl.ds(r0, RPT // WBC)], wb_buf)
        pltpu.sync_copy(wb_buf, cnt_out.at[cid, pl.ds(r0, RPT // WBC)])
        return carry

    lax.fori_loop(0, WBC, wb, 0)


@functools.cache
def _sc_gather_kernel():
    return pl.kernel(
        _sc_gather_body,
        out_type=(
            jax.ShapeDtypeStruct((ECH, DN), jnp.float32),
            jax.ShapeDtypeStruct((ECH, DN), jnp.float32),
        ),
        mesh=_mesh(),
        scratch_types=[
            pltpu.VMEM((CH,), jnp.int32),
            pltpu.VMEM((CH,), jnp.int32),
            pltpu.VMEM((CH,), jnp.int32),
            pltpu.VMEM((CH,), jnp.int32),
            pltpu.VMEM((CH, DN), jnp.float32),
            pltpu.VMEM((CH, DN), jnp.float32),
            pltpu.VMEM((CH, DN), jnp.float32),
            pltpu.VMEM((CH, DN), jnp.float32),
            pltpu.SemaphoreType.DMA,
            pltpu.SemaphoreType.DMA,
            pltpu.SemaphoreType.DMA,
            pltpu.SemaphoreType.DMA,
        ],
    )


def _sc_gather_body(nodes_hbm, r_hbm, s_hbm, a_out, c_out,
                    ri0, si0, ri1, si1, a0, c0, a1, c1, sg0, sg1, sw0, sw1):
    cid = lax.axis_index("c")
    sid = lax.axis_index("s")
    wid = cid * NS + sid
    base = wid * EWC
    nch = EWC // CH

    def start_g(j, ri, si, ab, cb, sg):
        off = base + j * CH
        pltpu.sync_copy(r_hbm.at[pl.ds(off, CH)], ri)
        pltpu.sync_copy(s_hbm.at[pl.ds(off, CH)], si)
        pltpu.async_copy(nodes_hbm.at[ri], ab, sg)
        pltpu.async_copy(nodes_hbm.at[si], cb, sg)

    def wait_g(ri, si, ab, cb, sg):
        pltpu.make_async_copy(nodes_hbm.at[ri], ab, sg).wait()
        pltpu.make_async_copy(nodes_hbm.at[si], cb, sg).wait()

    def start_w(j, ab, cb, sw):
        off = base + j * CH
        pltpu.async_copy(ab, a_out.at[pl.ds(off, CH)], sw)
        pltpu.async_copy(cb, c_out.at[pl.ds(off, CH)], sw)

    def wait_w(ab, cb, sw):
        pltpu.make_async_copy(ab, a_out.at[pl.ds(0, CH)], sw).wait()
        pltpu.make_async_copy(cb, c_out.at[pl.ds(0, CH)], sw).wait()

    start_g(0, ri0, si0, a0, c0, sg0)
    start_g(1, ri1, si1, a1, c1, sg1)

    def body(t, carry):
        j = 2 * t
        wait_g(ri0, si0, a0, c0, sg0)
        start_w(j, a0, c0, sw0)
        wait_g(ri1, si1, a1, c1, sg1)
        start_w(j + 1, a1, c1, sw1)
        wait_w(a0, c0, sw0)
        start_g(j + 2, ri0, si0, a0, c0, sg0)
        wait_w(a1, c1, sw1)

        @pl.when(t < (nch - 1) // 2 - 1)
        def _():
            start_g(j + 3, ri1, si1, a1, c1, sg1)

        return carry

    lax.fori_loop(0, (nch - 1) // 2, body, 0)
    wait_g(ri0, si0, a0, c0, sg0)
    start_w(nch - 1, a0, c0, sw0)
    wait_w(a0, c0, sw0)


@functools.cache
def _sc_scatter_kernel(ks):
    return pl.kernel(
        functools.partial(_sc_scatter_body, ks),
        out_type=jax.ShapeDtypeStruct((NC, NPAD, DN), jnp.float32),
        mesh=_mesh(),
        scratch_types=[
            pltpu.VMEM((CH,), jnp.int32),
            pltpu.VMEM((CH,), jnp.int32),
            pltpu.VMEM((CH, DN), jnp.float32),
            pltpu.VMEM((CH, DN), jnp.float32),
            pltpu.VMEM((RPT // WBC, DN), jnp.float32),
            pltpu.VMEM_SHARED((NPAD, DN), jnp.float32),
            pltpu.SemaphoreType.DMA,
            pltpu.SemaphoreType.DMA,
            pltpu.SemaphoreType.DMA,
            pltpu.SemaphoreType.DMA,
        ],
    )


def _sc_scatter_body(ks, *refs):
    e2s = refs[:len(ks)]
    (r_hbm, zeros_hbm, p_out,
     i0, i1, b0, b1, wb_buf, acc, sl0, sl1, sa0, sa1) = refs[len(ks):]
    cid = lax.axis_index("c")
    sid = lax.axis_index("s")
    wid = cid * NS + sid
    tb = sid * RPT
    nch = EWC // CH

    pltpu.sync_copy(zeros_hbm, acc.at[pl.ds(tb, RPT)])
    plsc.subcore_barrier()

    for k, e2_hbm in zip(ks, e2s):
        gbase = k * ECH + wid * EWC
        lbase = wid * EWC

        def start_l(j, ib, rb, sl, e2_hbm=e2_hbm, gbase=gbase, lbase=lbase):
            pltpu.async_copy(r_hbm.at[pl.ds(gbase + j * CH, CH)], ib, sl)
            pltpu.async_copy(e2_hbm.at[pl.ds(lbase + j * CH, CH)], rb, sl)

        def wait_l(ib, rb, sl):
            pltpu.make_async_copy(r_hbm.at[pl.ds(0, CH)], ib, sl).wait()
            pltpu.make_async_copy(e20.at[pl.ds(0, CH)], rb, sl).wait()

        def start_a(ib, rb, sa):
            pltpu.async_copy(rb, acc.at[ib], sa, add=True)

        def wait_a(rb, sa):
            pltpu.make_async_copy(rb, acc.at[pl.ds(0, CH)], sa).wait()

        def start_l(j, ib, rb, sl, e2_hbm=e2_hbm, gbase=gbase, lbase=lbase):
            pltpu.async_copy(r_hbm.at[pl.ds(gbase + j * CH, CH)], ib, sl)
            pltpu.async_copy(e2_hbm.at[pl.ds(lbase + j * CH, CH)], rb, sl)

        def wait_l(ib, rb, sl, e2_hbm=e2_hbm, gbase=gbase, lbase=lbase):
            pltpu.make_async_copy(r_hbm.at[pl.ds(gbase, CH)], ib, sl).wait()
            pltpu.make_async_copy(e2_hbm.at[pl.ds(lbase, CH)], rb, sl).wait()

        def start_a(ib, rb, sa):
            pltpu.async_copy(rb, acc.at[ib], sa, add=True)

        def wait_a(ib, rb, sa):
            pltpu.make_async_copy(rb, acc.at[ib], sa).wait()

        start_l(0, i0, b0, sl0)
        start_l(1, i1, b1, sl1)

        def body(t, carry):
            j = 2 * t
            wait_l(i0, b0, sl0)
            start_a(i0, b0, sa0)
            wait_l(i1, b1, sl1)
            start_a(i1, b1, sa1)
            wait_a(i0, b0, sa0)
            start_l(j + 2, i0, b0, sl0)
            wait_a(i1, b1, sa1)

            @pl.when(t < (nch - 1) // 2 - 1)
            def _():
                start_l(j + 3, i1, b1, sl1)

            return carry

        lax.fori_loop(0, (nch - 1) // 2, body, 0)
        wait_l(i0, b0, sl0)
        start_a(i0, b0, sa0)
        wait_a(i0, b0, sa0)
    plsc.subcore_barrier()

    def wb(k, carry):
        r0 = tb + k * (RPT // WBC)
        pltpu.sync_copy(acc.at[pl.ds(r0, RPT // WBC)], wb_buf)
        pltpu.sync_copy(wb_buf, p_out.at[cid, pl.ds(r0, RPT // WBC)])
        return carry

    lax.fori_loop(0, WBC, wb, 0)


def _mlp_body(a_ref, c_ref, e_ref, w1_ref, b1_ref, w2_ref, b2_ref,
              w3_ref, b3_ref, e2_ref, eo_ref):
    w1 = w1_ref[...]
    h = jnp.dot(a_ref[...].astype(jnp.bfloat16), w1[0:DN, :],
                preferred_element_type=jnp.float32)
    h += jnp.dot(e_ref[...].astype(jnp.bfloat16), w1[DN:DN + DE, :],
                 preferred_element_type=jnp.float32)
    h += jnp.dot(c_ref[...].astype(jnp.bfloat16), w1[DN + DE:, :],
                 preferred_element_type=jnp.float32)
    h = jax.nn.relu(h + b1_ref[...])
    e2 = jax.nn.relu(
        jnp.dot(h.astype(jnp.bfloat16), w2_ref[...],
                preferred_element_type=jnp.float32) + b2_ref[...])
    e2_ref[...] = e2
    eo_ref[...] = jax.nn.relu(
        jnp.dot(e2.astype(jnp.bfloat16), w3_ref[...],
                preferred_element_type=jnp.float32) + b3_ref[...])


def _tc_mlp(a, c, e, w1, b1, w2, b2, w3, b3, te=1280):
    grid = ECH // te
    blk = lambda d: pl.BlockSpec((te, d), lambda i: (i, 0))
    full = lambda s: pl.BlockSpec(s, lambda i: (0,) * len(s))
    return pl.pallas_call(
        _mlp_body,
        grid=(grid,),
        in_specs=[
            blk(DN), blk(DN), blk(DE),
            full((DN + DE + DN, H1)), full((1, H1)),
            full((H1, DN)), full((1, DN)),
            full((DN, DE)), full((1, DE)),
        ],
        out_specs=[blk(DN), blk(DE)],
        out_shape=[
            jax.ShapeDtypeStruct((ECH, DN), jnp.float32),
            jax.ShapeDtypeStruct((ECH, DE), jnp.float32),
        ],
    )(a, c, e, w1, b1, w2, b2, w3, b3)


def _combine_body(pa_ref, pb_ref, cnt_ref, o_ref):
    s = (pa_ref[0, 0:N, :] + pa_ref[1, 0:N, :]
         + pb_ref[0, 0:N, :] + pb_ref[1, 0:N, :])
    cnt = cnt_ref[0, 0:N, 0:1] + cnt_ref[1, 0:N, 0:1]
    o_ref[...] = s / jnp.maximum(cnt, 1.0)


def _tc_combine(pa, pb, cnt):
    return pl.pallas_call(
        _combine_body,
        out_shape=jax.ShapeDtypeStruct((N, DN), jnp.float32),
    )(pa, pb, cnt)


def kernel(nodes, edges, senders, receivers, W1, b1, W2, b2, W3, b3):
    b = nodes.shape[0]
    nodes_flat = nodes.reshape(N, DN)
    edges_flat = edges.reshape(E, DE)
    r = receivers.reshape(E)
    s = senders.reshape(E)

    zeros = jnp.zeros((RPT, DN), jnp.float32)
    ones = jnp.ones((CH, DN), jnp.float32)
    cnt = _sc_counts_kernel()(r, zeros, ones)

    w1b = W1.astype(jnp.bfloat16)
    w2b = W2.astype(jnp.bfloat16)
    w3b = W3.astype(jnp.bfloat16)
    b1r, b2r, b3r = b1.reshape(1, H1), b2.reshape(1, DN), b3.reshape(1, DE)

    e2s, eos = [], []
    for k in range(NCH):
        sl = slice(k * ECH, (k + 1) * ECH)
        a_k, c_k = _sc_gather_kernel()(nodes_flat, r[sl], s[sl])
        e2_k, eo_k = _tc_mlp(a_k, c_k, edges_flat[sl],
                             w1b, b1r, w2b, b2r, w3b, b3r)
        e2s.append(e2_k)
        eos.append(eo_k)
    edges_out = jnp.concatenate(eos, axis=0)

    pa = _sc_scatter_kernel((0, 1, 2))(*e2s[0:3], r, zeros)
    pb = _sc_scatter_kernel((3, 4))(*e2s[3:5], r, zeros)
    nodes_out = _tc_combine(pa, pb, cnt)
    return (nodes_out.reshape(b, N, DN), edges_out.reshape(b, E, DE),
            senders, receivers)
